# baseline (speedup 1.0000x reference)
import functools

import jax
import jax.numpy as jnp
from jax import lax
from jax.experimental import pallas as pl
from jax.experimental.pallas import tpu as pltpu
from jax.experimental.pallas import tpu_sc as plsc

EPS = 1e-6
LANES = 16
NC, NS = 2, 16
NW = NC * NS
CHUNK = 16


def _rsqrt16(v16):
    ii = lax.bitcast_convert_type(v16, jnp.int32)
    yi = jnp.int32(0x5F3759DF) - lax.shift_right_arithmetic(ii, 1)
    y = lax.bitcast_convert_type(yi, jnp.float32)
    for _ in range(3):
        y = y * (1.5 - 0.5 * v16 * y * y)
    return y


@functools.lru_cache(maxsize=None)
def _build_sc_call(n_rows, seq, d):
    rpw = n_rows // NW
    n_chunks = rpw // CHUNK
    nv = d // LANES
    inv_d = 1.0 / d
    mesh = plsc.VectorSubcoreMesh(core_axis_name="c", subcore_axis_name="s")

    @functools.partial(
        pl.kernel,
        mesh=mesh,
        compiler_params=pltpu.CompilerParams(needs_layout_passes=False),
        out_type=jax.ShapeDtypeStruct((n_rows, d), jnp.float32),
        scratch_types=[
            pltpu.VMEM((CHUNK,), jnp.int32),
            pltpu.VMEM((CHUNK, d), jnp.float32),
            pltpu.VMEM((CHUNK, d), jnp.float32),
            pltpu.VMEM((CHUNK, d), jnp.float32),
            pltpu.VMEM((d,), jnp.float32),
            pltpu.VMEM((d,), jnp.float32),
            pltpu.SemaphoreType.DMA,
        ],
    )
    def sc_call(ids_hbm, tok_hbm, pos_hbm, gam_hbm, bet_hbm, out_hbm,
                idx_v, tok_v, pos_v, out_v, gam_v, bet_v, sem):
        wid = lax.axis_index("s") * NC + lax.axis_index("c")
        base = wid * rpw
        pos_base = lax.rem(base, seq)
        pltpu.sync_copy(gam_hbm, gam_v)
        pltpu.sync_copy(bet_hbm, bet_v)

        def chunk_body(ci, _):
            r0 = base + ci * CHUNK
            p0 = pos_base + ci * CHUNK
            pltpu.sync_copy(ids_hbm.at[pl.ds(r0, CHUNK)], idx_v)
            gather = pltpu.async_copy(tok_hbm.at[idx_v], tok_v, sem)
            pltpu.sync_copy(pos_hbm.at[pl.ds(p0, CHUNK)], pos_v)
            gather.wait()

            def row_body(r, _):
                def acc_body(j, carry):
                    sm, sq = carry
                    o = j * LANES
                    x = tok_v[r, pl.ds(o, LANES)] + pos_v[r, pl.ds(o, LANES)]
                    out_v[r, pl.ds(o, LANES)] = x
                    return (sm + x, sq + x * x)

                zero = jnp.zeros((LANES,), jnp.float32)
                sm, sq = lax.fori_loop(0, nv, acc_body, (zero, zero))
                mean = jnp.sum(sm) * inv_d
                ex2 = jnp.sum(sq) * inv_d
                var = ex2 - mean * mean
                rstd = _rsqrt16(jnp.full((LANES,), var + EPS, jnp.float32))
                shift = jnp.full((LANES,), mean, jnp.float32) * rstd

                def norm_body(j, _):
                    o = j * LANES
                    xn = out_v[r, pl.ds(o, LANES)] * rstd - shift
                    out_v[r, pl.ds(o, LANES)] = (
                        xn * gam_v[pl.ds(o, LANES)] + bet_v[pl.ds(o, LANES)])
                    return 0

                lax.fori_loop(0, nv, norm_body, 0)
                return 0

            lax.fori_loop(0, CHUNK, row_body, 0)
            pltpu.sync_copy(out_v, out_hbm.at[pl.ds(r0, CHUNK)])
            return 0

        lax.fori_loop(0, n_chunks, chunk_body, 0)

    return sc_call


def kernel(input_ids, token_table, pos_table, ln_gamma, ln_beta):
    b, s = input_ids.shape
    d = token_table.shape[1]
    ids = input_ids.reshape(-1).astype(jnp.int32)
    sc_call = _build_sc_call(b * s, s, d)
    out = sc_call(ids, token_table, pos_table, ln_gamma, ln_beta)
    return out.reshape(b, s, d)


# speedup vs baseline: 1.3494x; 1.3494x over previous
import functools

import jax
import jax.numpy as jnp
from jax import lax
from jax.experimental import pallas as pl
from jax.experimental.pallas import tpu as pltpu
from jax.experimental.pallas import tpu_sc as plsc

EPS = 1e-6
LANES = 16
NC, NS = 2, 16
NW = NC * NS
CHUNK = 16
UNROLL = 4


def _rsqrt16(v16):
    ii = lax.bitcast_convert_type(v16, jnp.int32)
    yi = jnp.int32(0x5F3759DF) - lax.shift_right_arithmetic(ii, 1)
    y = lax.bitcast_convert_type(yi, jnp.float32)
    for _ in range(3):
        y = y * (1.5 - 0.5 * v16 * y * y)
    return y


@functools.lru_cache(maxsize=None)
def _build_sc_call(n_rows, seq, d):
    rpw = n_rows // NW
    n_chunks = rpw // CHUNK
    n_pairs = n_chunks // 2
    nv = d // LANES
    n_steps = nv // UNROLL
    inv_d = 1.0 / d
    mesh = plsc.VectorSubcoreMesh(core_axis_name="c", subcore_axis_name="s")

    @functools.partial(
        pl.kernel,
        mesh=mesh,
        compiler_params=pltpu.CompilerParams(needs_layout_passes=False),
        out_type=jax.ShapeDtypeStruct((n_rows, d), jnp.float32),
        scratch_types=[
            pltpu.VMEM((rpw,), jnp.int32),
            pltpu.VMEM((CHUNK, d), jnp.float32),
            pltpu.VMEM((CHUNK, d), jnp.float32),
            pltpu.VMEM((CHUNK, d), jnp.float32),
            pltpu.VMEM((CHUNK, d), jnp.float32),
            pltpu.VMEM((CHUNK, d), jnp.float32),
            pltpu.VMEM((CHUNK, d), jnp.float32),
            pltpu.VMEM((d,), jnp.float32),
            pltpu.VMEM((d,), jnp.float32),
            pltpu.SemaphoreType.DMA,
            pltpu.SemaphoreType.DMA,
            pltpu.SemaphoreType.DMA,
            pltpu.SemaphoreType.DMA,
            pltpu.SemaphoreType.DMA,
            pltpu.SemaphoreType.DMA,
        ],
    )
    def sc_call(ids_hbm, tok_hbm, pos_hbm, gam_hbm, bet_hbm, out_hbm,
                idx_v, tok0, tok1, pos0, pos1, out0, out1, gam_v, bet_v,
                gs0, gs1, ps0, ps1, os0, os1):
        wid = lax.axis_index("s") * NC + lax.axis_index("c")
        base = wid * rpw
        pos_base = lax.rem(base, seq)
        toks = (tok0, tok1)
        poss = (pos0, pos1)
        outs = (out0, out1)
        gsems = (gs0, gs1)
        psems = (ps0, ps1)
        osems = (os0, os1)

        pltpu.sync_copy(ids_hbm.at[pl.ds(base, rpw)], idx_v)
        pltpu.sync_copy(gam_hbm, gam_v)
        pltpu.sync_copy(bet_hbm, bet_v)

        def issue_in(ci, b):
            pltpu.async_copy(
                tok_hbm.at[idx_v.at[pl.ds(ci * CHUNK, CHUNK)]], toks[b],
                gsems[b])
            pltpu.async_copy(
                pos_hbm.at[pl.ds(pos_base + ci * CHUNK, CHUNK)], poss[b],
                psems[b])

        def wait_in(ci, b):
            pltpu.make_async_copy(
                tok_hbm.at[idx_v.at[pl.ds(ci * CHUNK, CHUNK)]], toks[b],
                gsems[b]).wait()
            pltpu.make_async_copy(
                pos_hbm.at[pl.ds(pos_base + ci * CHUNK, CHUNK)], poss[b],
                psems[b]).wait()

        def wait_out(ci, b):
            pltpu.make_async_copy(
                outs[b], out_hbm.at[pl.ds(base + ci * CHUNK, CHUNK)],
                osems[b]).wait()

        issue_in(0, 0)
        issue_in(1, 1)

        def compute_chunk(tok_v, pos_v, out_v):
            zero = jnp.zeros((LANES,), jnp.float32)
            for r in range(CHUNK):
                def acc_body(j, carry):
                    s0, s1, s2, s3, q0, q1, q2, q3 = carry
                    o = j * (LANES * UNROLL)
                    x0 = tok_v[r, pl.ds(o, LANES)] + pos_v[r, pl.ds(o, LANES)]
                    x1 = (tok_v[r, pl.ds(o + 16, LANES)]
                          + pos_v[r, pl.ds(o + 16, LANES)])
                    x2 = (tok_v[r, pl.ds(o + 32, LANES)]
                          + pos_v[r, pl.ds(o + 32, LANES)])
                    x3 = (tok_v[r, pl.ds(o + 48, LANES)]
                          + pos_v[r, pl.ds(o + 48, LANES)])
                    out_v[r, pl.ds(o, LANES)] = x0
                    out_v[r, pl.ds(o + 16, LANES)] = x1
                    out_v[r, pl.ds(o + 32, LANES)] = x2
                    out_v[r, pl.ds(o + 48, LANES)] = x3
                    return (s0 + x0, s1 + x1, s2 + x2, s3 + x3,
                            q0 + x0 * x0, q1 + x1 * x1, q2 + x2 * x2,
                            q3 + x3 * x3)

                s0, s1, s2, s3, q0, q1, q2, q3 = lax.fori_loop(
                    0, n_steps, acc_body, (zero,) * 8)
                sm = (s0 + s1) + (s2 + s3)
                sq = (q0 + q1) + (q2 + q3)
                mean = jnp.sum(sm) * inv_d
                ex2 = jnp.sum(sq) * inv_d
                var = ex2 - mean * mean
                scale = _rsqrt16(jnp.full((LANES,), var + EPS, jnp.float32))
                shift = jnp.full((LANES,), mean, jnp.float32) * scale

                def norm_body(j, _):
                    o = j * (LANES * UNROLL)
                    for k in range(UNROLL):
                        ok = o + k * LANES
                        xn = out_v[r, pl.ds(ok, LANES)] * scale - shift
                        out_v[r, pl.ds(ok, LANES)] = (
                            xn * gam_v[pl.ds(ok, LANES)]
                            + bet_v[pl.ds(ok, LANES)])
                    return 0

                lax.fori_loop(0, n_steps, norm_body, 0)

        def pair_body(cp, _):
            for b in (0, 1):
                ci = cp * 2 + b
                wait_in(ci, b)

                @pl.when(cp > 0)
                def _():
                    wait_out(ci - 2, b)

                compute_chunk(toks[b], poss[b], outs[b])
                pltpu.async_copy(
                    outs[b], out_hbm.at[pl.ds(base + ci * CHUNK, CHUNK)],
                    osems[b])

                @pl.when(cp < n_pairs - 1)
                def _():
                    issue_in(ci + 2, b)
            return 0

        lax.fori_loop(0, n_pairs, pair_body, 0)
        wait_out(n_chunks - 2, 0)
        wait_out(n_chunks - 1, 1)

    return sc_call


def kernel(input_ids, token_table, pos_table, ln_gamma, ln_beta):
    b, s = input_ids.shape
    d = token_table.shape[1]
    ids = input_ids.reshape(-1).astype(jnp.int32)
    sc_call = _build_sc_call(b * s, s, d)
    out = sc_call(ids, token_table, pos_table, ln_gamma, ln_beta)
    return out.reshape(b, s, d)


# speedup vs baseline: 3.6787x; 2.7261x over previous
import functools

import jax
import jax.numpy as jnp
from jax import lax
from jax.experimental import pallas as pl
from jax.experimental.pallas import tpu as pltpu
from jax.experimental.pallas import tpu_sc as plsc

EPS = 1e-6
LANES = 16
NC, NS = 2, 16
NW = NC * NS
CHUNK = 16
UNROLL = 4


def _rsqrt16(v16):
    ii = lax.bitcast_convert_type(v16, jnp.int32)
    yi = jnp.int32(0x5F3759DF) - lax.shift_right_arithmetic(ii, 1)
    y = lax.bitcast_convert_type(yi, jnp.float32)
    for _ in range(3):
        y = y * (1.5 - 0.5 * v16 * y * y)
    return y


@functools.lru_cache(maxsize=None)
def _build_sc_call(n_rows, seq, d):
    rpw = n_rows // NW
    n_chunks = rpw // CHUNK
    n_pairs = n_chunks // 2
    nv = d // LANES
    n_steps = nv // UNROLL
    inv_d = 1.0 / d
    mesh = plsc.VectorSubcoreMesh(core_axis_name="c", subcore_axis_name="s")

    @functools.partial(
        pl.kernel,
        mesh=mesh,
        compiler_params=pltpu.CompilerParams(needs_layout_passes=False),
        out_type=jax.ShapeDtypeStruct((n_rows, d), jnp.float32),
        scratch_types=[
            pltpu.VMEM((rpw,), jnp.int32),
            pltpu.VMEM((CHUNK, d), jnp.float32),
            pltpu.VMEM((CHUNK, d), jnp.float32),
            pltpu.VMEM((CHUNK, d), jnp.float32),
            pltpu.VMEM((CHUNK, d), jnp.float32),
            pltpu.VMEM((CHUNK, d), jnp.float32),
            pltpu.VMEM((CHUNK, d), jnp.float32),
            pltpu.VMEM((d,), jnp.float32),
            pltpu.VMEM((d,), jnp.float32),
            pltpu.SemaphoreType.DMA,
            pltpu.SemaphoreType.DMA,
            pltpu.SemaphoreType.DMA,
            pltpu.SemaphoreType.DMA,
            pltpu.SemaphoreType.DMA,
            pltpu.SemaphoreType.DMA,
        ],
    )
    def sc_call(ids_hbm, tok_hbm, pos_hbm, gam_hbm, bet_hbm, out_hbm,
                idx_v, tok0, tok1, pos0, pos1, out0, out1, gam_v, bet_v,
                gs0, gs1, ps0, ps1, os0, os1):
        wid = lax.axis_index("s") * NC + lax.axis_index("c")
        base = wid * rpw
        pos_base = lax.rem(base, seq)
        toks = (tok0, tok1)
        poss = (pos0, pos1)
        outs = (out0, out1)
        gsems = (gs0, gs1)
        psems = (ps0, ps1)
        osems = (os0, os1)

        pltpu.sync_copy(ids_hbm.at[pl.ds(base, rpw)], idx_v)
        pltpu.sync_copy(gam_hbm, gam_v)
        pltpu.sync_copy(bet_hbm, bet_v)

        def issue_in(ci, b):
            pltpu.async_copy(
                tok_hbm.at[idx_v.at[pl.ds(ci * CHUNK, CHUNK)]], toks[b],
                gsems[b])
            pltpu.async_copy(
                pos_hbm.at[pl.ds(pos_base + ci * CHUNK, CHUNK)], poss[b],
                psems[b])

        def wait_in(ci, b):
            pltpu.make_async_copy(
                tok_hbm.at[idx_v.at[pl.ds(ci * CHUNK, CHUNK)]], toks[b],
                gsems[b]).wait()
            pltpu.make_async_copy(
                pos_hbm.at[pl.ds(pos_base + ci * CHUNK, CHUNK)], poss[b],
                psems[b]).wait()

        def wait_out(ci, b):
            pltpu.make_async_copy(
                outs[b], out_hbm.at[pl.ds(base + ci * CHUNK, CHUNK)],
                osems[b]).wait()

        issue_in(0, 0)
        issue_in(1, 1)

        def compute_chunk(tok_v, pos_v, out_v):
            zero = jnp.zeros((LANES,), jnp.float32)

            def row_body(r, _):
                @plsc.parallel_loop(0, nv, 1, unroll=UNROLL,
                                    carry=(zero, zero))
                def sums(j, carry):
                    sm, sq = carry
                    o = j * LANES
                    x = tok_v[r, pl.ds(o, LANES)] + pos_v[r, pl.ds(o, LANES)]
                    out_v[r, pl.ds(o, LANES)] = x
                    return (sm + x, sq + x * x)

                sm, sq = sums
                mean = jnp.sum(sm) * inv_d
                ex2 = jnp.sum(sq) * inv_d
                var = ex2 - mean * mean
                scale = _rsqrt16(jnp.full((LANES,), var + EPS, jnp.float32))
                shift = jnp.full((LANES,), mean, jnp.float32) * scale

                @plsc.parallel_loop(0, nv, 1, unroll=UNROLL)
                def _(j):
                    o = j * LANES
                    xn = out_v[r, pl.ds(o, LANES)] * scale - shift
                    out_v[r, pl.ds(o, LANES)] = (
                        xn * gam_v[pl.ds(o, LANES)] + bet_v[pl.ds(o, LANES)])

                return 0

            lax.fori_loop(0, CHUNK, row_body, 0)

        def pair_body(cp, _):
            for b in (0, 1):
                ci = cp * 2 + b
                wait_in(ci, b)

                @pl.when(cp > 0)
                def _():
                    wait_out(ci - 2, b)

                compute_chunk(toks[b], poss[b], outs[b])
                pltpu.async_copy(
                    outs[b], out_hbm.at[pl.ds(base + ci * CHUNK, CHUNK)],
                    osems[b])

                @pl.when(cp < n_pairs - 1)
                def _():
                    issue_in(ci + 2, b)
            return 0

        lax.fori_loop(0, n_pairs, pair_body, 0)
        wait_out(n_chunks - 2, 0)
        wait_out(n_chunks - 1, 1)

    return sc_call


def kernel(input_ids, token_table, pos_table, ln_gamma, ln_beta):
    b, s = input_ids.shape
    d = token_table.shape[1]
    ids = input_ids.reshape(-1).astype(jnp.int32)
    sc_call = _build_sc_call(b * s, s, d)
    out = sc_call(ids, token_table, pos_table, ln_gamma, ln_beta)
    return out.reshape(b, s, d)


# speedup vs baseline: 3.7043x; 1.0070x over previous
import functools

import jax
import jax.numpy as jnp
from jax import lax
from jax.experimental import pallas as pl
from jax.experimental.pallas import tpu as pltpu
from jax.experimental.pallas import tpu_sc as plsc

EPS = 1e-6
LANES = 16
NC, NS = 2, 16
NW = NC * NS
CHUNK = 16
UNROLL = 8


def _rsqrt16(v16):
    ii = lax.bitcast_convert_type(v16, jnp.int32)
    yi = jnp.int32(0x5F3759DF) - lax.shift_right_arithmetic(ii, 1)
    y = lax.bitcast_convert_type(yi, jnp.float32)
    for _ in range(3):
        y = y * (1.5 - 0.5 * v16 * y * y)
    return y


@functools.lru_cache(maxsize=None)
def _build_sc_call(n_rows, seq, d):
    rpw = n_rows // NW
    n_chunks = rpw // CHUNK
    n_pairs = n_chunks // 2
    nv = d // LANES
    n_steps = nv // UNROLL
    inv_d = 1.0 / d
    mesh = plsc.VectorSubcoreMesh(core_axis_name="c", subcore_axis_name="s")

    @functools.partial(
        pl.kernel,
        mesh=mesh,
        compiler_params=pltpu.CompilerParams(needs_layout_passes=False),
        out_type=jax.ShapeDtypeStruct((n_rows, d), jnp.float32),
        scratch_types=[
            pltpu.VMEM((rpw,), jnp.int32),
            pltpu.VMEM((CHUNK, d), jnp.float32),
            pltpu.VMEM((CHUNK, d), jnp.float32),
            pltpu.VMEM((CHUNK, d), jnp.float32),
            pltpu.VMEM((CHUNK, d), jnp.float32),
            pltpu.VMEM((CHUNK, d), jnp.float32),
            pltpu.VMEM((CHUNK, d), jnp.float32),
            pltpu.VMEM((d,), jnp.float32),
            pltpu.VMEM((d,), jnp.float32),
            pltpu.SemaphoreType.DMA,
            pltpu.SemaphoreType.DMA,
            pltpu.SemaphoreType.DMA,
            pltpu.SemaphoreType.DMA,
            pltpu.SemaphoreType.DMA,
            pltpu.SemaphoreType.DMA,
        ],
    )
    def sc_call(ids_hbm, tok_hbm, pos_hbm, gam_hbm, bet_hbm, out_hbm,
                idx_v, tok0, tok1, pos0, pos1, out0, out1, gam_v, bet_v,
                gs0, gs1, ps0, ps1, os0, os1):
        wid = lax.axis_index("s") * NC + lax.axis_index("c")
        base = wid * rpw
        pos_base = lax.rem(base, seq)
        toks = (tok0, tok1)
        poss = (pos0, pos1)
        outs = (out0, out1)
        gsems = (gs0, gs1)
        psems = (ps0, ps1)
        osems = (os0, os1)

        pltpu.sync_copy(ids_hbm.at[pl.ds(base, rpw)], idx_v)
        pltpu.sync_copy(gam_hbm, gam_v)
        pltpu.sync_copy(bet_hbm, bet_v)

        def issue_in(ci, b):
            pltpu.async_copy(
                tok_hbm.at[idx_v.at[pl.ds(ci * CHUNK, CHUNK)]], toks[b],
                gsems[b])
            pltpu.async_copy(
                pos_hbm.at[pl.ds(pos_base + ci * CHUNK, CHUNK)], poss[b],
                psems[b])

        def wait_in(ci, b):
            pltpu.make_async_copy(
                tok_hbm.at[idx_v.at[pl.ds(ci * CHUNK, CHUNK)]], toks[b],
                gsems[b]).wait()
            pltpu.make_async_copy(
                pos_hbm.at[pl.ds(pos_base + ci * CHUNK, CHUNK)], poss[b],
                psems[b]).wait()

        def wait_out(ci, b):
            pltpu.make_async_copy(
                outs[b], out_hbm.at[pl.ds(base + ci * CHUNK, CHUNK)],
                osems[b]).wait()

        issue_in(0, 0)
        issue_in(1, 1)

        def compute_chunk(tok_v, pos_v, out_v):
            zero = jnp.zeros((LANES,), jnp.float32)

            def row_body(r, _):
                @plsc.parallel_loop(0, nv, 1, unroll=UNROLL,
                                    carry=(zero, zero))
                def sums(j, carry):
                    sm, sq = carry
                    o = j * LANES
                    x = tok_v[r, pl.ds(o, LANES)] + pos_v[r, pl.ds(o, LANES)]
                    out_v[r, pl.ds(o, LANES)] = x
                    return (sm + x, sq + x * x)

                sm, sq = sums
                mean = jnp.sum(sm) * inv_d
                ex2 = jnp.sum(sq) * inv_d
                var = ex2 - mean * mean
                scale = _rsqrt16(jnp.full((LANES,), var + EPS, jnp.float32))
                shift = jnp.full((LANES,), mean, jnp.float32) * scale

                @plsc.parallel_loop(0, nv, 1, unroll=UNROLL)
                def _(j):
                    o = j * LANES
                    xn = out_v[r, pl.ds(o, LANES)] * scale - shift
                    out_v[r, pl.ds(o, LANES)] = (
                        xn * gam_v[pl.ds(o, LANES)] + bet_v[pl.ds(o, LANES)])

                return 0

            lax.fori_loop(0, CHUNK, row_body, 0)

        def pair_body(cp, _):
            for b in (0, 1):
                ci = cp * 2 + b
                wait_in(ci, b)

                @pl.when(cp > 0)
                def _():
                    wait_out(ci - 2, b)

                compute_chunk(toks[b], poss[b], outs[b])
                pltpu.async_copy(
                    outs[b], out_hbm.at[pl.ds(base + ci * CHUNK, CHUNK)],
                    osems[b])

                @pl.when(cp < n_pairs - 1)
                def _():
                    issue_in(ci + 2, b)
            return 0

        lax.fori_loop(0, n_pairs, pair_body, 0)
        wait_out(n_chunks - 2, 0)
        wait_out(n_chunks - 1, 1)

    return sc_call


def kernel(input_ids, token_table, pos_table, ln_gamma, ln_beta):
    b, s = input_ids.shape
    d = token_table.shape[1]
    ids = input_ids.reshape(-1).astype(jnp.int32)
    sc_call = _build_sc_call(b * s, s, d)
    out = sc_call(ids, token_table, pos_table, ln_gamma, ln_beta)
    return out.reshape(b, s, d)


# speedup vs baseline: 4.3920x; 1.1856x over previous
import functools

import jax
import jax.numpy as jnp
from jax import lax
from jax.experimental import pallas as pl
from jax.experimental.pallas import tpu as pltpu
from jax.experimental.pallas import tpu_sc as plsc

EPS = 1e-6
LANES = 16
NC, NS = 2, 16
NW = NC * NS
CHUNK = 16
UNROLL = 8
RB = 4


def _rsqrt_scalar(v):
    ii = lax.bitcast_convert_type(v, jnp.int32)
    yi = jnp.int32(0x5F3759DF) - lax.shift_right_arithmetic(ii, 1)
    y = lax.bitcast_convert_type(yi, jnp.float32)
    for _ in range(3):
        y = y * (1.5 - 0.5 * v * y * y)
    return y


@functools.lru_cache(maxsize=None)
def _build_sc_call(n_rows, seq, d):
    rpw = n_rows // NW
    n_chunks = rpw // CHUNK
    n_pairs = n_chunks // 2
    nv = d // LANES
    n_steps = nv // UNROLL
    inv_d = 1.0 / d
    mesh = plsc.VectorSubcoreMesh(core_axis_name="c", subcore_axis_name="s")

    @functools.partial(
        pl.kernel,
        mesh=mesh,
        compiler_params=pltpu.CompilerParams(needs_layout_passes=False),
        out_type=jax.ShapeDtypeStruct((n_rows, d), jnp.float32),
        scratch_types=[
            pltpu.VMEM((rpw,), jnp.int32),
            pltpu.VMEM((CHUNK, d), jnp.float32),
            pltpu.VMEM((CHUNK, d), jnp.float32),
            pltpu.VMEM((CHUNK, d), jnp.float32),
            pltpu.VMEM((CHUNK, d), jnp.float32),
            pltpu.VMEM((CHUNK, d), jnp.float32),
            pltpu.VMEM((CHUNK, d), jnp.float32),
            pltpu.VMEM((d,), jnp.float32),
            pltpu.VMEM((d,), jnp.float32),
            pltpu.SMEM((CHUNK,), jnp.float32),
            pltpu.SMEM((CHUNK,), jnp.float32),
            pltpu.SemaphoreType.DMA,
            pltpu.SemaphoreType.DMA,
            pltpu.SemaphoreType.DMA,
            pltpu.SemaphoreType.DMA,
            pltpu.SemaphoreType.DMA,
            pltpu.SemaphoreType.DMA,
        ],
    )
    def sc_call(ids_hbm, tok_hbm, pos_hbm, gam_hbm, bet_hbm, out_hbm,
                idx_v, tok0, tok1, pos0, pos1, out0, out1, gam_v, bet_v,
                scale_s, shift_s, gs0, gs1, ps0, ps1, os0, os1):
        wid = lax.axis_index("s") * NC + lax.axis_index("c")
        base = wid * rpw
        pos_base = lax.rem(base, seq)
        toks = (tok0, tok1)
        poss = (pos0, pos1)
        outs = (out0, out1)
        gsems = (gs0, gs1)
        psems = (ps0, ps1)
        osems = (os0, os1)

        pltpu.sync_copy(ids_hbm.at[pl.ds(base, rpw)], idx_v)
        pltpu.sync_copy(gam_hbm, gam_v)
        pltpu.sync_copy(bet_hbm, bet_v)

        def issue_in(ci, b):
            pltpu.async_copy(
                tok_hbm.at[idx_v.at[pl.ds(ci * CHUNK, CHUNK)]], toks[b],
                gsems[b])
            pltpu.async_copy(
                pos_hbm.at[pl.ds(pos_base + ci * CHUNK, CHUNK)], poss[b],
                psems[b])

        def wait_in(ci, b):
            pltpu.make_async_copy(
                tok_hbm.at[idx_v.at[pl.ds(ci * CHUNK, CHUNK)]], toks[b],
                gsems[b]).wait()
            pltpu.make_async_copy(
                pos_hbm.at[pl.ds(pos_base + ci * CHUNK, CHUNK)], poss[b],
                psems[b]).wait()

        def wait_out(ci, b):
            pltpu.make_async_copy(
                outs[b], out_hbm.at[pl.ds(base + ci * CHUNK, CHUNK)],
                osems[b]).wait()

        issue_in(0, 0)
        issue_in(1, 1)

        def compute_chunk(tok_v, pos_v, out_v):
            zero = jnp.zeros((LANES,), jnp.float32)

            def row_body(r, _):
                @plsc.parallel_loop(0, nv, 1, unroll=UNROLL,
                                    carry=(zero, zero))
                def sums(j, carry):
                    sm, sq = carry
                    o = j * LANES
                    x = tok_v[r, pl.ds(o, LANES)] + pos_v[r, pl.ds(o, LANES)]
                    out_v[r, pl.ds(o, LANES)] = x
                    return (sm + x, sq + x * x)

                sm, sq = sums
                mean = jnp.sum(sm) * inv_d
                ex2 = jnp.sum(sq) * inv_d
                var = ex2 - mean * mean
                rstd = _rsqrt_scalar(var + EPS)
                scale_s[r] = rstd
                shift_s[r] = mean * rstd
                return 0

            lax.fori_loop(0, CHUNK, row_body, 0)

            for rb in range(0, CHUNK, RB):
                scs = [jnp.full((LANES,), scale_s[rb + k], jnp.float32)
                       for k in range(RB)]
                shs = [jnp.full((LANES,), shift_s[rb + k], jnp.float32)
                       for k in range(RB)]

                @plsc.parallel_loop(0, nv, 1, unroll=UNROLL)
                def _(j):
                    o = j * LANES
                    g = gam_v[pl.ds(o, LANES)]
                    b = bet_v[pl.ds(o, LANES)]
                    for k in range(RB):
                        x = out_v[rb + k, pl.ds(o, LANES)]
                        out_v[rb + k, pl.ds(o, LANES)] = (
                            (x * scs[k] - shs[k]) * g + b)

        def pair_body(cp, _):
            for b in (0, 1):
                ci = cp * 2 + b
                wait_in(ci, b)

                @pl.when(cp > 0)
                def _():
                    wait_out(ci - 2, b)

                compute_chunk(toks[b], poss[b], outs[b])
                pltpu.async_copy(
                    outs[b], out_hbm.at[pl.ds(base + ci * CHUNK, CHUNK)],
                    osems[b])

                @pl.when(cp < n_pairs - 1)
                def _():
                    issue_in(ci + 2, b)
            return 0

        lax.fori_loop(0, n_pairs, pair_body, 0)
        wait_out(n_chunks - 2, 0)
        wait_out(n_chunks - 1, 1)

    return sc_call


def kernel(input_ids, token_table, pos_table, ln_gamma, ln_beta):
    b, s = input_ids.shape
    d = token_table.shape[1]
    ids = input_ids.reshape(-1).astype(jnp.int32)
    sc_call = _build_sc_call(b * s, s, d)
    out = sc_call(ids, token_table, pos_table, ln_gamma, ln_beta)
    return out.reshape(b, s, d)


# speedup vs baseline: 4.5939x; 1.0460x over previous
import functools

import jax
import jax.numpy as jnp
from jax import lax
from jax.experimental import pallas as pl
from jax.experimental.pallas import tpu as pltpu
from jax.experimental.pallas import tpu_sc as plsc

EPS = 1e-6
LANES = 16
NC, NS = 2, 16
NW = NC * NS
CHUNK = 16
UNROLL = 8
RB = 4


def _rsqrt_scalar(v):
    ii = lax.bitcast_convert_type(v, jnp.int32)
    yi = jnp.int32(0x5F3759DF) - lax.shift_right_arithmetic(ii, 1)
    y = lax.bitcast_convert_type(yi, jnp.float32)
    for _ in range(3):
        y = y * (1.5 - 0.5 * v * y * y)
    return y


@functools.lru_cache(maxsize=None)
def _build_sc_call(n_rows, seq, d):
    rpw = n_rows // NW
    n_chunks = rpw // CHUNK
    n_pairs = n_chunks // 2
    nv = d // LANES
    n_steps = nv // UNROLL
    inv_d = 1.0 / d
    mesh = plsc.VectorSubcoreMesh(core_axis_name="c", subcore_axis_name="s")

    @functools.partial(
        pl.kernel,
        mesh=mesh,
        compiler_params=pltpu.CompilerParams(needs_layout_passes=False),
        out_type=jax.ShapeDtypeStruct((n_rows, d), jnp.float32),
        scratch_types=[
            pltpu.VMEM((rpw,), jnp.int32),
            pltpu.VMEM((CHUNK, d), jnp.float32),
            pltpu.VMEM((CHUNK, d), jnp.float32),
            pltpu.VMEM((CHUNK, d), jnp.float32),
            pltpu.VMEM((CHUNK, d), jnp.float32),
            pltpu.VMEM((CHUNK, d), jnp.float32),
            pltpu.VMEM((CHUNK, d), jnp.float32),
            pltpu.VMEM((d,), jnp.float32),
            pltpu.VMEM((d,), jnp.float32),
            pltpu.SMEM((CHUNK,), jnp.float32),
            pltpu.SMEM((CHUNK,), jnp.float32),
            pltpu.VMEM((CHUNK, LANES), jnp.float32),
            pltpu.VMEM((CHUNK, LANES), jnp.float32),
            pltpu.SemaphoreType.DMA,
            pltpu.SemaphoreType.DMA,
            pltpu.SemaphoreType.DMA,
            pltpu.SemaphoreType.DMA,
            pltpu.SemaphoreType.DMA,
            pltpu.SemaphoreType.DMA,
        ],
    )
    def sc_call(ids_hbm, tok_hbm, pos_hbm, gam_hbm, bet_hbm, out_hbm,
                idx_v, tok0, tok1, pos0, pos1, out0, out1, gam_v, bet_v,
                scale_s, shift_s, smv, sqv, gs0, gs1, ps0, ps1, os0, os1):
        wid = lax.axis_index("s") * NC + lax.axis_index("c")
        base = wid * rpw
        pos_base = lax.rem(base, seq)
        toks = (tok0, tok1)
        poss = (pos0, pos1)
        outs = (out0, out1)
        gsems = (gs0, gs1)
        psems = (ps0, ps1)
        osems = (os0, os1)

        pltpu.sync_copy(ids_hbm.at[pl.ds(base, rpw)], idx_v)
        pltpu.sync_copy(gam_hbm, gam_v)
        pltpu.sync_copy(bet_hbm, bet_v)

        def issue_in(ci, b):
            pltpu.async_copy(
                tok_hbm.at[idx_v.at[pl.ds(ci * CHUNK, CHUNK)]], toks[b],
                gsems[b])
            pltpu.async_copy(
                pos_hbm.at[pl.ds(pos_base + ci * CHUNK, CHUNK)], poss[b],
                psems[b])

        def wait_in(ci, b):
            pltpu.make_async_copy(
                tok_hbm.at[idx_v.at[pl.ds(ci * CHUNK, CHUNK)]], toks[b],
                gsems[b]).wait()
            pltpu.make_async_copy(
                pos_hbm.at[pl.ds(pos_base + ci * CHUNK, CHUNK)], poss[b],
                psems[b]).wait()

        def wait_out(ci, b):
            pltpu.make_async_copy(
                outs[b], out_hbm.at[pl.ds(base + ci * CHUNK, CHUNK)],
                osems[b]).wait()

        issue_in(0, 0)
        issue_in(1, 1)

        def compute_chunk(tok_v, pos_v, out_v):
            zero = jnp.zeros((LANES,), jnp.float32)

            def row_body(r, _):
                @plsc.parallel_loop(0, nv, 1, unroll=UNROLL,
                                    carry=(zero, zero))
                def sums(j, carry):
                    sm, sq = carry
                    o = j * LANES
                    x = tok_v[r, pl.ds(o, LANES)] + pos_v[r, pl.ds(o, LANES)]
                    out_v[r, pl.ds(o, LANES)] = x
                    return (sm + x, sq + x * x)

                sm, sq = sums
                smv[r] = sm
                sqv[r] = sq
                return 0

            lax.fori_loop(0, CHUNK, row_body, 0)

            for r in range(CHUNK):
                mean = jnp.sum(smv[r]) * inv_d
                ex2 = jnp.sum(sqv[r]) * inv_d
                var = ex2 - mean * mean
                rstd = _rsqrt_scalar(var + EPS)
                scale_s[r] = rstd
                shift_s[r] = mean * rstd

            for rb in range(0, CHUNK, RB):
                scs = [jnp.full((LANES,), scale_s[rb + k], jnp.float32)
                       for k in range(RB)]
                shs = [jnp.full((LANES,), shift_s[rb + k], jnp.float32)
                       for k in range(RB)]

                @plsc.parallel_loop(0, nv, 1, unroll=UNROLL)
                def _(j):
                    o = j * LANES
                    g = gam_v[pl.ds(o, LANES)]
                    b = bet_v[pl.ds(o, LANES)]
                    for k in range(RB):
                        x = out_v[rb + k, pl.ds(o, LANES)]
                        out_v[rb + k, pl.ds(o, LANES)] = (
                            (x * scs[k] - shs[k]) * g + b)

        def pair_body(cp, _):
            for b in (0, 1):
                ci = cp * 2 + b
                wait_in(ci, b)

                @pl.when(cp > 0)
                def _():
                    wait_out(ci - 2, b)

                compute_chunk(toks[b], poss[b], outs[b])
                pltpu.async_copy(
                    outs[b], out_hbm.at[pl.ds(base + ci * CHUNK, CHUNK)],
                    osems[b])

                @pl.when(cp < n_pairs - 1)
                def _():
                    issue_in(ci + 2, b)
            return 0

        lax.fori_loop(0, n_pairs, pair_body, 0)
        wait_out(n_chunks - 2, 0)
        wait_out(n_chunks - 1, 1)

    return sc_call


def kernel(input_ids, token_table, pos_table, ln_gamma, ln_beta):
    b, s = input_ids.shape
    d = token_table.shape[1]
    ids = input_ids.reshape(-1).astype(jnp.int32)
    sc_call = _build_sc_call(b * s, s, d)
    out = sc_call(ids, token_table, pos_table, ln_gamma, ln_beta)
    return out.reshape(b, s, d)
